# baseline (speedup 1.0000x reference)
import functools

import jax
import jax.numpy as jnp
import numpy as np
from jax import lax
from jax.experimental import pallas as pl
from jax.experimental.pallas import tpu as pltpu
from jax.experimental.pallas import tpu_sc as plsc

_FIELD_DIM = 38462
_NUM_FIELDS = 26
_EMBED_DIM = 16
_BATCH = 16384

_NC = 2
_NS = 16
_NW = _NC * _NS

_ROWS = _BATCH * _NUM_FIELDS
_PER_W = _ROWS // _NW
_CHUNK = 1664
_NCHUNK = _PER_W // _CHUNK
_SUB = 128
_NSUB = _CHUNK // _SUB
_PERIOD = 208

_OFF_PATTERN = np.asarray(
    (np.arange(_PERIOD) % _NUM_FIELDS) * _FIELD_DIM, dtype=np.int32
)

_mesh = plsc.VectorSubcoreMesh(core_axis_name="c", subcore_axis_name="s")


@functools.partial(
    pl.kernel,
    mesh=_mesh,
    out_type=jax.ShapeDtypeStruct((_ROWS, _EMBED_DIM), jnp.float32),
    compiler_params=pltpu.CompilerParams(use_tc_tiling_on_sc=False),
    scratch_types=[
        pltpu.VMEM((_PERIOD,), jnp.int32),
        pltpu.VMEM((_CHUNK,), jnp.int32),
        pltpu.VMEM((_CHUNK, _EMBED_DIM), jnp.float32),
        pltpu.SemaphoreType.DMA,
    ],
)
def _sc_gather(x_hbm, off_hbm, table_hbm, out_hbm, off_v, idx_v, rows_v, sem):
    wid = lax.axis_index("s") * _NC + lax.axis_index("c")
    wbase = wid * _PER_W

    pltpu.sync_copy(off_hbm, off_v)

    def chunk_body(c, carry):
        base = wbase + c * _CHUNK
        pltpu.sync_copy(x_hbm.at[pl.ds(base, _CHUNK)], idx_v)

        def off_body(g, carry2):
            gb = g * _PERIOD
            for j in range(_PERIOD // 16):
                s = pl.ds(gb + j * 16, 16)
                idx_v[s] = idx_v[s] + off_v[pl.ds(j * 16, 16)]
            return carry2

        lax.fori_loop(0, _CHUNK // _PERIOD, off_body, 0, unroll=False)

        copies = [
            pltpu.async_copy(
                table_hbm.at[idx_v.at[pl.ds(j * _SUB, _SUB)]],
                rows_v.at[pl.ds(j * _SUB, _SUB)],
                sem,
            )
            for j in range(_NSUB)
        ]
        for cp in copies:
            cp.wait()

        pltpu.sync_copy(rows_v, out_hbm.at[pl.ds(base, _CHUNK)])
        return carry

    lax.fori_loop(0, _NCHUNK, chunk_body, 0, unroll=False)


def kernel(x, table):
    off = jnp.asarray(_OFF_PATTERN)
    out = _sc_gather(x.reshape(_ROWS), off, table)
    return out.reshape(_BATCH, _NUM_FIELDS, _EMBED_DIM)


# speedup vs baseline: 1.0001x; 1.0001x over previous
import functools

import jax
import jax.numpy as jnp
import numpy as np
from jax import lax
from jax.experimental import pallas as pl
from jax.experimental.pallas import tpu as pltpu
from jax.experimental.pallas import tpu_sc as plsc

_FIELD_DIM = 38462
_NUM_FIELDS = 26
_EMBED_DIM = 16
_BATCH = 16384

_NC = 2
_NS = 16
_NW = _NC * _NS

_ROWS = _BATCH * _NUM_FIELDS
_PER_W = _ROWS // _NW
_CHUNK = 1664
_NCHUNK = _PER_W // _CHUNK
_SUB = 128
_NSUB = _CHUNK // _SUB
_PERIOD = 208
_ROWS_TOTAL = _FIELD_DIM * _NUM_FIELDS * _EMBED_DIM

_OFF_PATTERN = np.asarray(
    (np.arange(_PERIOD) % _NUM_FIELDS) * _FIELD_DIM, dtype=np.int32
)

_mesh = plsc.VectorSubcoreMesh(core_axis_name="c", subcore_axis_name="s")


@functools.partial(
    pl.kernel,
    mesh=_mesh,
    out_type=jax.ShapeDtypeStruct((_ROWS, _EMBED_DIM), jnp.float32),
    compiler_params=pltpu.CompilerParams(use_tc_tiling_on_sc=False),
    scratch_types=[
        pltpu.VMEM((_PERIOD,), jnp.int32),
        pltpu.VMEM((_CHUNK,), jnp.int32),
        pltpu.VMEM((_CHUNK, _EMBED_DIM), jnp.float32),
        pltpu.SemaphoreType.DMA,
    ],
)
def _sc_gather(x_hbm, off_hbm, table_hbm, out_hbm, off_v, idx_v, rows_v, sem):
    wid = lax.axis_index("s") * _NC + lax.axis_index("c")
    wbase = wid * _PER_W

    pltpu.sync_copy(off_hbm, off_v)

    def chunk_body(c, carry):
        base = wbase + c * _CHUNK
        pltpu.sync_copy(x_hbm.at[pl.ds(base, _CHUNK)], idx_v)

        def off_body(g, carry2):
            gb = g * _PERIOD
            for j in range(_PERIOD // 16):
                s = pl.ds(gb + j * 16, 16)
                idx_v[s] = idx_v[s] + off_v[pl.ds(j * 16, 16)]
            return carry2

        lax.fori_loop(0, _CHUNK // _PERIOD, off_body, 0, unroll=False)

        copies = [
            pltpu.async_copy(
                table_hbm.at[idx_v.at[pl.ds(j * _SUB, _SUB)]],
                rows_v.at[pl.ds(j * _SUB, _SUB)],
                sem,
            )
            for j in range(_NSUB)
        ]
        for cp in copies:
            cp.wait()

        pltpu.sync_copy(rows_v, out_hbm.at[pl.ds(base, _CHUNK)])
        return carry

    lax.fori_loop(0, _NCHUNK, chunk_body, 0, unroll=False)


def kernel(x, table):
    off = jnp.asarray(_OFF_PATTERN)
    tbl_lin = jax.lax.optimization_barrier(table.reshape(_ROWS_TOTAL))
    tbl = tbl_lin.reshape(_FIELD_DIM * _NUM_FIELDS, _EMBED_DIM)
    out = _sc_gather(x.reshape(_ROWS), off, tbl)
    return out.reshape(_BATCH, _NUM_FIELDS, _EMBED_DIM)


# speedup vs baseline: 1.6003x; 1.6002x over previous
import functools

import jax
import jax.numpy as jnp
from jax import lax
from jax.experimental import pallas as pl
from jax.experimental.pallas import tpu as pltpu
from jax.experimental.pallas import tpu_sc as plsc

_FIELD_DIM = 38462
_NUM_FIELDS = 26
_EMBED_DIM = 16
_BATCH = 16384

_NC = 2
_NS = 16
_NW = _NC * _NS

_ROWS = _BATCH * _NUM_FIELDS
_VOCAB = _FIELD_DIM * _NUM_FIELDS
_BTILES = _BATCH // 128
_PAIRS = _NUM_FIELDS * _BTILES
_PAIRS_W = _PAIRS // _NW
_CP = 8
_CHUNK = _CP * 128
_NCHUNK = _PAIRS_W // _CP
_SUB = 128
_NSUB = _CHUNK // _SUB

_mesh = plsc.VectorSubcoreMesh(core_axis_name="c", subcore_axis_name="s")


@functools.partial(
    pl.kernel,
    mesh=_mesh,
    out_type=jax.ShapeDtypeStruct(
        (_NUM_FIELDS, 2, _BTILES, 8, 128), jnp.float32
    ),
    compiler_params=pltpu.CompilerParams(
        use_tc_tiling_on_sc=False, needs_layout_passes=False
    ),
    scratch_types=[
        pltpu.VMEM((_CHUNK,), jnp.int32),
        pltpu.VMEM((_CHUNK, _EMBED_DIM), jnp.float32),
        pltpu.VMEM((2, _CP, 8, 128), jnp.float32),
        pltpu.SemaphoreType.DMA,
    ],
)
def _sc_gather(x_hbm, table_hbm, out_hbm, idx_v, rows_v, tile_v, sem):
    wid = lax.axis_index("s") * _NC + lax.axis_index("c")
    wpair = wid * _PAIRS_W
    lane = lax.iota(jnp.int32, 16)

    def chunk_body(c, carry):
        p0 = wpair + c * _CP
        f = p0 // _BTILES
        tc0 = p0 % _BTILES
        b0 = tc0 * 128

        pltpu.sync_copy(x_hbm.at[pl.ds(f * _BATCH + b0, _CHUNK)], idx_v)

        off = f * _FIELD_DIM

        def off_body(i, carry2):
            s = pl.ds(i * 16, 16)
            idx_v[s] = idx_v[s] + off
            return carry2

        lax.fori_loop(0, _CHUNK // 16, off_body, 0, unroll=False)

        copies = [
            pltpu.async_copy(
                table_hbm.at[idx_v.at[pl.ds(j * _SUB, _SUB)]],
                rows_v.at[pl.ds(j * _SUB, _SUB)],
                sem,
            )
            for j in range(_NSUB)
        ]
        for cp in copies:
            cp.wait()

        def tp_body(t, carry2):
            for tr in range(2):
                for r in range(8):
                    col = jnp.full((16,), tr * 8 + r, jnp.int32)
                    for cb in range(8):
                        row = t * 128 + cb * 16 + lane
                        v = plsc.load_gather(rows_v, [row, col])
                        tile_v[tr, t, r, pl.ds(cb * 16, 16)] = v
            return carry2

        lax.fori_loop(0, _CP, tp_body, 0, unroll=False)

        pltpu.sync_copy(tile_v.at[0], out_hbm.at[f, 0, pl.ds(tc0, _CP)])
        pltpu.sync_copy(tile_v.at[1], out_hbm.at[f, 1, pl.ds(tc0, _CP)])
        return carry

    lax.fori_loop(0, _NCHUNK, chunk_body, 0, unroll=False)


def kernel(x, table):
    x_fm = x.T.reshape(_ROWS)
    tbl_lin = jax.lax.optimization_barrier(
        table.reshape(_VOCAB * _EMBED_DIM)
    )
    tbl = tbl_lin.reshape(_VOCAB, _EMBED_DIM)
    out5 = _sc_gather(x_fm, tbl)
    return out5.transpose(2, 4, 0, 1, 3).reshape(_BATCH, _NUM_FIELDS, _EMBED_DIM)


# speedup vs baseline: 1.7292x; 1.0805x over previous
import functools

import jax
import jax.numpy as jnp
from jax import lax
from jax.experimental import pallas as pl
from jax.experimental.pallas import tpu as pltpu
from jax.experimental.pallas import tpu_sc as plsc

_FIELD_DIM = 38462
_NUM_FIELDS = 26
_EMBED_DIM = 16
_BATCH = 16384

_NC = 2
_NS = 16
_NW = _NC * _NS

_ROWS = _BATCH * _NUM_FIELDS
_VOCAB = _FIELD_DIM * _NUM_FIELDS
_BTILES = _BATCH // 128
_PAIRS = _NUM_FIELDS * _BTILES
_PAIRS_W = _PAIRS // _NW
_CP = 8
_CHUNK = _CP * 128
_NCHUNK = _PAIRS_W // _CP
_SUB = 128
_NSUB = _CHUNK // _SUB

_mesh = plsc.VectorSubcoreMesh(core_axis_name="c", subcore_axis_name="s")

_TCOLS = _VOCAB // 128
_TAIL = _VOCAB - _TCOLS * 128
_BLK_W = (_TCOLS + _NW - 1) // _NW


@functools.partial(
    pl.kernel,
    mesh=_mesh,
    out_type=jax.ShapeDtypeStruct((_VOCAB * _EMBED_DIM,), jnp.float32),
    compiler_params=pltpu.CompilerParams(
        use_tc_tiling_on_sc=True, needs_layout_passes=False
    ),
    scratch_types=[
        pltpu.VMEM((2, 2, 8, 128), jnp.float32),
        pltpu.VMEM((2, 128 * _EMBED_DIM), jnp.float32),
        pltpu.VMEM((_TAIL * _EMBED_DIM,), jnp.float32),
        pltpu.SemaphoreType.DMA,
        pltpu.SemaphoreType.DMA,
        pltpu.SemaphoreType.DMA,
        pltpu.SemaphoreType.DMA,
    ],
)
def _sc_relayout(tbl_t, tail, out_flat, in_v, row_v, tail_v, sem_in0, sem_in1, sem_out0, sem_out1):
    sem_in = (sem_in0, sem_in1)
    sem_out = (sem_out0, sem_out1)
    wid = lax.axis_index("s") * _NC + lax.axis_index("c")
    lane = lax.iota(jnp.int32, 16)
    tr_idx = lane // 8
    r_idx = lane % 8
    nblk = (_TCOLS - wid + _NW - 1) // _NW

    @pl.when(wid == 0)
    def _():
        pltpu.sync_copy(tail, tail_v)
        pltpu.sync_copy(
            tail_v, out_flat.at[pl.ds(_TCOLS * 128 * _EMBED_DIM, _TAIL * _EMBED_DIM)]
        )

    def fire_in(blk, buf):
        tc = wid + blk * _NW

        @pl.when(blk < nblk)
        def _():
            for tr in range(2):
                pltpu.async_copy(
                    tbl_t.at[pl.ds(tr * 8, 8), pl.ds(tc * 128, 128)],
                    in_v.at[buf, tr],
                    sem_in[buf],
                )

    def wait_in(blk, buf):
        @pl.when(blk < nblk)
        def _():
            for tr in range(2):
                pltpu.make_async_copy(
                    tbl_t.at[pl.ds(0, 8), pl.ds(0, 128)], in_v.at[buf, tr], sem_in[buf]
                ).wait()

    def wait_out(blk, buf):
        @pl.when(jnp.logical_and(blk >= 0, blk < nblk))
        def _():
            pltpu.make_async_copy(
                row_v.at[buf], out_flat.at[pl.ds(0, 128 * _EMBED_DIM)], sem_out[buf]
            ).wait()

    fire_in(0, 0)
    fire_in(1, 1)

    def pair_body(k, carry):
        for half in range(2):
            blk = k * 2 + half

            @pl.when(blk < nblk)
            def _(blk=blk, half=half):
                tc = wid + blk * _NW
                wait_in(blk, half)
                wait_out(blk - 2, half)

                def col_body(g, carry2):
                    for j in range(16):
                        c = g * 16 + j
                        cv = jnp.full((16,), c, jnp.int32)
                        v = plsc.load_gather(in_v.at[half], [tr_idx, r_idx, cv])
                        row_v[half, pl.ds(c * 16, 16)] = v
                    return carry2

                lax.fori_loop(0, 8, col_body, 0, unroll=False)
                pltpu.async_copy(
                    row_v.at[half],
                    out_flat.at[
                        pl.ds((wid + blk * _NW) * 128 * _EMBED_DIM, 128 * _EMBED_DIM)
                    ],
                    sem_out[half],
                )
                fire_in(blk + 2, half)

        return carry

    lax.fori_loop(0, (_BLK_W + 1) // 2, pair_body, 0, unroll=False)

    @pl.when(nblk % 2 == 0)
    def _():
        wait_out(nblk - 2, 0)
        wait_out(nblk - 1, 1)

    @pl.when(nblk % 2 == 1)
    def _():
        wait_out(nblk - 2, 1)
        wait_out(nblk - 1, 0)


@functools.partial(
    pl.kernel,
    mesh=_mesh,
    out_type=jax.ShapeDtypeStruct(
        (_NUM_FIELDS, 2, _BTILES, 8, 128), jnp.float32
    ),
    compiler_params=pltpu.CompilerParams(
        use_tc_tiling_on_sc=False, needs_layout_passes=False
    ),
    scratch_types=[
        pltpu.VMEM((_CHUNK,), jnp.int32),
        pltpu.VMEM((_CHUNK, _EMBED_DIM), jnp.float32),
        pltpu.VMEM((2, _CP, 8, 128), jnp.float32),
        pltpu.SemaphoreType.DMA,
    ],
)
def _sc_gather(x_hbm, table_hbm, out_hbm, idx_v, rows_v, tile_v, sem):
    wid = lax.axis_index("s") * _NC + lax.axis_index("c")
    wpair = wid * _PAIRS_W
    lane = lax.iota(jnp.int32, 16)

    def chunk_body(c, carry):
        p0 = wpair + c * _CP
        f = p0 // _BTILES
        tc0 = p0 % _BTILES
        b0 = tc0 * 128

        pltpu.sync_copy(x_hbm.at[pl.ds(f * _BATCH + b0, _CHUNK)], idx_v)

        off = f * _FIELD_DIM

        def off_body(i, carry2):
            s = pl.ds(i * 16, 16)
            idx_v[s] = idx_v[s] + off
            return carry2

        lax.fori_loop(0, _CHUNK // 16, off_body, 0, unroll=False)

        copies = [
            pltpu.async_copy(
                table_hbm.at[idx_v.at[pl.ds(j * _SUB, _SUB)]],
                rows_v.at[pl.ds(j * _SUB, _SUB)],
                sem,
            )
            for j in range(_NSUB)
        ]
        for cp in copies:
            cp.wait()

        def tp_body(t, carry2):
            for tr in range(2):
                for r in range(8):
                    col = jnp.full((16,), tr * 8 + r, jnp.int32)
                    for cb in range(8):
                        row = t * 128 + cb * 16 + lane
                        v = plsc.load_gather(rows_v, [row, col])
                        tile_v[tr, t, r, pl.ds(cb * 16, 16)] = v
            return carry2

        lax.fori_loop(0, _CP, tp_body, 0, unroll=False)

        pltpu.sync_copy(tile_v.at[0], out_hbm.at[f, 0, pl.ds(tc0, _CP)])
        pltpu.sync_copy(tile_v.at[1], out_hbm.at[f, 1, pl.ds(tc0, _CP)])
        return carry

    lax.fori_loop(0, _NCHUNK, chunk_body, 0, unroll=False)


def kernel(x, table):
    x_fm = x.T.reshape(_ROWS)
    tail = table[_TCOLS * 128 :].reshape(_TAIL * _EMBED_DIM)
    tbl_flat = _sc_relayout(table.T, tail)
    tbl = tbl_flat.reshape(_VOCAB, _EMBED_DIM)
    out5 = _sc_gather(x_fm, tbl)
    return out5.transpose(2, 4, 0, 1, 3).reshape(_BATCH, _NUM_FIELDS, _EMBED_DIM)


# speedup vs baseline: 1.7493x; 1.0116x over previous
import functools

import jax
import jax.numpy as jnp
from jax import lax
from jax.experimental import pallas as pl
from jax.experimental.pallas import tpu as pltpu
from jax.experimental.pallas import tpu_sc as plsc

_FIELD_DIM = 38462
_NUM_FIELDS = 26
_EMBED_DIM = 16
_BATCH = 16384

_NC = 2
_NS = 16
_NW = _NC * _NS

_ROWS = _BATCH * _NUM_FIELDS
_VOCAB = _FIELD_DIM * _NUM_FIELDS
_BTILES = _BATCH // 128
_PAIRS = _NUM_FIELDS * _BTILES
_PAIRS_W = _PAIRS // _NW
_CP = 8
_CHUNK = _CP * 128
_NCHUNK = _PAIRS_W // _CP
_SUB = 128
_NSUB = _CHUNK // _SUB

_mesh = plsc.VectorSubcoreMesh(core_axis_name="c", subcore_axis_name="s")

_GCOLS = 1024
_GSKEW = _GCOLS + 1
_FULLCOLS = (_VOCAB // 128) * 128
_NGRP = _FULLCOLS // _GCOLS
_REM = _FULLCOLS - _NGRP * _GCOLS
_TAIL = _VOCAB - _FULLCOLS
_GRP_W = (_NGRP + _NW - 1) // _NW


@functools.partial(
    pl.kernel,
    mesh=_mesh,
    out_type=jax.ShapeDtypeStruct((_VOCAB * _EMBED_DIM,), jnp.float32),
    compiler_params=pltpu.CompilerParams(
        use_tc_tiling_on_sc=True, needs_layout_passes=False
    ),
    scratch_types=[
        pltpu.VMEM((2, _EMBED_DIM, _GSKEW), jnp.float32),
        pltpu.VMEM((2, _GCOLS * _EMBED_DIM), jnp.float32),
        pltpu.VMEM((_TAIL * _EMBED_DIM,), jnp.float32),
        pltpu.SemaphoreType.DMA,
        pltpu.SemaphoreType.DMA,
        pltpu.SemaphoreType.DMA,
        pltpu.SemaphoreType.DMA,
    ],
)
def _sc_relayout(
    tbl_t, tail, out_flat, in_v, row_v, tail_v, sem_in0, sem_in1, sem_out0, sem_out1
):
    sem_in = (sem_in0, sem_in1)
    sem_out = (sem_out0, sem_out1)
    wid = lax.axis_index("s") * _NC + lax.axis_index("c")
    lane = lax.iota(jnp.int32, 16)
    nblk = (_NGRP - wid + _NW - 1) // _NW

    def transpose_cols(buf, ngroups):
        def col_body(g, carry2):
            cb = jnp.full((16,), g * 16, jnp.int32)
            for j in range(16):
                v = plsc.load_gather(in_v.at[buf], [lane, cb + j])
                row_v[buf, pl.ds(g * 256 + j * 16, 16)] = v
            return carry2

        lax.fori_loop(0, ngroups, col_body, 0, unroll=False)

    @pl.when(wid == 0)
    def _():
        pltpu.sync_copy(tail, tail_v)
        pltpu.sync_copy(
            tail_v,
            out_flat.at[
                pl.ds(_VOCAB * _EMBED_DIM - _TAIL * _EMBED_DIM, _TAIL * _EMBED_DIM)
            ],
        )

    @pl.when(wid == _NW - 1)
    def _():
        for tr in range(2):
            pltpu.sync_copy(
                tbl_t.at[pl.ds(tr * 8, 8), pl.ds(_NGRP * _GCOLS, _REM)],
                in_v.at[0, pl.ds(tr * 8, 8), pl.ds(0, _REM)],
            )
        transpose_cols(0, _REM // 16)
        pltpu.sync_copy(
            row_v.at[0, pl.ds(0, _REM * _EMBED_DIM)],
            out_flat.at[pl.ds(_NGRP * _GCOLS * _EMBED_DIM, _REM * _EMBED_DIM)],
        )

    def fire_in(blk, buf):
        @pl.when(blk < nblk)
        def _():
            c0 = (wid + blk * _NW) * _GCOLS
            for tr in range(2):
                pltpu.async_copy(
                    tbl_t.at[pl.ds(tr * 8, 8), pl.ds(c0, _GCOLS)],
                    in_v.at[buf, pl.ds(tr * 8, 8), pl.ds(0, _GCOLS)],
                    sem_in[buf],
                )

    def wait_in(blk, buf):
        @pl.when(blk < nblk)
        def _():
            for tr in range(2):
                pltpu.make_async_copy(
                    tbl_t.at[pl.ds(0, 8), pl.ds(0, _GCOLS)],
                    in_v.at[buf, pl.ds(0, 8), pl.ds(0, _GCOLS)],
                    sem_in[buf],
                ).wait()

    def wait_out(blk, buf):
        @pl.when(jnp.logical_and(blk >= 0, blk < nblk))
        def _():
            pltpu.make_async_copy(
                row_v.at[buf],
                out_flat.at[pl.ds(0, _GCOLS * _EMBED_DIM)],
                sem_out[buf],
            ).wait()

    fire_in(0, 0)
    fire_in(1, 1)

    def pair_body(k, carry):
        for half in range(2):
            blk = k * 2 + half

            @pl.when(blk < nblk)
            def _(blk=blk, half=half):
                wait_in(blk, half)
                wait_out(blk - 2, half)
                transpose_cols(half, _GCOLS // 16)
                pltpu.async_copy(
                    row_v.at[half],
                    out_flat.at[
                        pl.ds(
                            (wid + blk * _NW) * _GCOLS * _EMBED_DIM,
                            _GCOLS * _EMBED_DIM,
                        )
                    ],
                    sem_out[half],
                )
                fire_in(blk + 2, half)

        return carry

    lax.fori_loop(0, (_GRP_W + 1) // 2, pair_body, 0, unroll=False)

    @pl.when(nblk % 2 == 0)
    def _():
        wait_out(nblk - 2, 0)
        wait_out(nblk - 1, 1)

    @pl.when(nblk % 2 == 1)
    def _():
        wait_out(nblk - 2, 1)
        wait_out(nblk - 1, 0)


_RSKEW = _EMBED_DIM + 1


@functools.partial(
    pl.kernel,
    mesh=_mesh,
    out_type=jax.ShapeDtypeStruct(
        (_NUM_FIELDS, 2, _BTILES * 8 * 128), jnp.float32
    ),
    compiler_params=pltpu.CompilerParams(
        use_tc_tiling_on_sc=False, needs_layout_passes=False
    ),
    scratch_types=[
        pltpu.VMEM((_CHUNK,), jnp.int32),
        pltpu.VMEM((_CHUNK, _EMBED_DIM), jnp.float32),
        pltpu.VMEM((2 * _CP * 8 * 128,), jnp.float32),
        pltpu.SemaphoreType.DMA,
    ],
)
def _sc_gather(x_hbm, table_hbm, out_hbm, idx_v, rows_v, tile_v, sem):
    wid = lax.axis_index("s") * _NC + lax.axis_index("c")
    wpair = wid * _PAIRS_W
    lane = lax.iota(jnp.int32, 16)
    colvs, waddrs = [], []
    for d in range(16):
        e = (d + lane) % 16
        colvs.append(e)
        waddrs.append((e // 8) * (_CP * 8 * 128) + (e % 8) * 128 + lane)

    def chunk_body(c, carry):
        p0 = wpair + c * _CP
        f = p0 // _BTILES
        tc0 = p0 % _BTILES
        b0 = tc0 * 128

        pltpu.sync_copy(x_hbm.at[pl.ds(f * _BATCH + b0, _CHUNK)], idx_v)

        off = f * _FIELD_DIM

        def off_body(i, carry2):
            s = pl.ds(i * 16, 16)
            idx_v[s] = idx_v[s] + off
            return carry2

        lax.fori_loop(0, _CHUNK // 16, off_body, 0, unroll=False)

        copies = [
            pltpu.async_copy(
                table_hbm.at[idx_v.at[pl.ds(j * _SUB, _SUB)]],
                rows_v.at[pl.ds(j * _SUB, _SUB)],
                sem,
            )
            for j in range(_NSUB)
        ]
        for cp in copies:
            cp.wait()

        def tp_body(tci, carry2):
            rb = jnp.full((16,), tci * 128, jnp.int32) + lane
            sb = jnp.full((16,), tci * 1024, jnp.int32)
            for cb in range(8):
                rowv = rb + cb * 16
                sv = sb + cb * 16
                for d in range(16):
                    v = plsc.load_gather(rows_v, [rowv, colvs[d]])
                    plsc.store_scatter(tile_v, [waddrs[d] + sv], v)
            return carry2

        lax.fori_loop(0, _CP, tp_body, 0, unroll=False)

        for tr in range(2):
            pltpu.sync_copy(
                tile_v.at[pl.ds(tr * _CP * 1024, _CP * 1024)],
                out_hbm.at[f, tr, pl.ds(tc0 * 1024, _CP * 1024)],
            )
        return carry

    lax.fori_loop(0, _NCHUNK, chunk_body, 0, unroll=False)


def kernel(x, table):
    x_fm = x.T.reshape(_ROWS)
    tail = table[_FULLCOLS:].reshape(_TAIL * _EMBED_DIM)
    tbl_flat = _sc_relayout(table.T, tail)
    tbl = tbl_flat.reshape(_VOCAB, _EMBED_DIM)
    out = _sc_gather(x_fm, tbl)
    out5 = out.reshape(_NUM_FIELDS, 2, _BTILES, 8, 128)
    return out5.transpose(2, 4, 0, 1, 3).reshape(_BATCH, _NUM_FIELDS, _EMBED_DIM)
